# baseline (speedup 1.0000x reference)
import functools

import jax
import jax.numpy as jnp
from jax import lax
from jax.experimental import pallas as pl
from jax.experimental.pallas import tpu as pltpu
from jax.experimental.pallas import tpu_sc as plsc

NC = 2
NS = 16
L = 16
NW = NC * NS

_MESH = dict(core_axis_name="c", subcore_axis_name="s",
             num_cores=NC, num_subcores=NS)
_SC_PARAMS = pltpu.CompilerParams(needs_layout_passes=False)


def _wid():
    return lax.axis_index("c") * NS + lax.axis_index("s")


def _make_deg(ET, NP):
    EW = ET // NW
    steps = EW // L

    @functools.partial(
        pl.kernel,
        out_type=jax.ShapeDtypeStruct((NW, NP), jnp.float32),
        mesh=plsc.VectorSubcoreMesh(**_MESH),
        compiler_params=_SC_PARAMS,
        scratch_types=[
            pltpu.VMEM((EW,), jnp.int32),
            pltpu.VMEM((NP,), jnp.float32),
            pltpu.SemaphoreType.DMA,
        ],
    )
    def deg_kernel(sd_hbm, out_hbm, sdv, acc, sem):
        w = _wid()
        sl_edges = pl.ds(w * EW, EW)
        one = jnp.ones((L,), jnp.float32)
        cps = [
            pltpu.async_copy(sd_hbm.at[sl_edges], sdv, sem),
        ]
        zero = jnp.zeros((L,), jnp.float32)

        @plsc.parallel_loop(0, NP // L, unroll=8)
        def _(i):
            acc[pl.ds(i * L, L)] = zero

        for cp in cps:
            cp.wait()

        @plsc.parallel_loop(0, steps, unroll=4)
        def _(i):
            sl = pl.ds(i * L, L)
            plsc.addupdate_scatter(acc, [sdv[sl] & 16383], one)

        pltpu.async_copy(acc, out_hbm.at[w], sem).wait()

    return deg_kernel


def _make_agg_norm(ET, NP, K):
    EW = ET // NW
    steps = EW // L
    NPK = NP * K

    @functools.partial(
        pl.kernel,
        out_type=(jax.ShapeDtypeStruct((NW, NPK), jnp.float32),
                  jax.ShapeDtypeStruct((ET,), jnp.float32)),
        mesh=plsc.VectorSubcoreMesh(**_MESH),
        compiler_params=_SC_PARAMS,
        scratch_types=[
            pltpu.VMEM((NPK,), jnp.float32),
            pltpu.VMEM((EW,), jnp.int32),
            pltpu.VMEM((EW,), jnp.float32),
            pltpu.VMEM((NP,), jnp.float32),
            pltpu.VMEM((NPK,), jnp.float32),
            pltpu.SemaphoreType.DMA,
        ],
    )
    def agg_norm_kernel(h_hbm, sd_hbm, dinv_hbm,
                        out_hbm, nrm_hbm, hv, sdv, wv, dv, acc, sem):
        w = _wid()
        sl_edges = pl.ds(w * EW, EW)
        cps = [
            pltpu.async_copy(h_hbm, hv, sem),
            pltpu.async_copy(sd_hbm.at[sl_edges], sdv, sem),
            pltpu.async_copy(dinv_hbm, dv, sem),
        ]
        zero = jnp.zeros((L,), jnp.float32)

        @plsc.parallel_loop(0, NPK // L, unroll=8)
        def _(i):
            acc[pl.ds(i * L, L)] = zero

        for cp in cps:
            cp.wait()

        @plsc.parallel_loop(0, steps, unroll=4)
        def _(i):
            sl = pl.ds(i * L, L)
            sd16 = sdv[sl]
            s16 = lax.shift_right_logical(sd16, 14)
            d16 = sd16 & 16383
            a = plsc.load_gather(dv, [s16])
            b = plsc.load_gather(dv, [d16])
            n16 = a * b
            wv[sl] = n16
            for k in range(K):
                g = plsc.load_gather(hv, [s16 + (k * NP)])
                plsc.addupdate_scatter(acc, [d16 + (k * NP)], g * n16)

        cp = pltpu.async_copy(wv, nrm_hbm.at[sl_edges], sem)
        pltpu.async_copy(acc, out_hbm.at[w], sem).wait()
        cp.wait()

    return agg_norm_kernel


def _make_agg(ET, NP, K):
    EW = ET // NW
    steps = EW // L
    NPK = NP * K

    @functools.partial(
        pl.kernel,
        out_type=jax.ShapeDtypeStruct((NW, NPK), jnp.float32),
        mesh=plsc.VectorSubcoreMesh(**_MESH),
        compiler_params=_SC_PARAMS,
        scratch_types=[
            pltpu.VMEM((NPK,), jnp.float32),
            pltpu.VMEM((EW,), jnp.int32),
            pltpu.VMEM((EW,), jnp.float32),
            pltpu.VMEM((NPK,), jnp.float32),
            pltpu.SemaphoreType.DMA,
        ],
    )
    def agg_kernel(h_hbm, sd_hbm, nrm_hbm, out_hbm,
                   hv, sdv, nrmv, acc, sem):
        w = _wid()
        sl_edges = pl.ds(w * EW, EW)
        cps = [
            pltpu.async_copy(h_hbm, hv, sem),
            pltpu.async_copy(sd_hbm.at[sl_edges], sdv, sem),
            pltpu.async_copy(nrm_hbm.at[sl_edges], nrmv, sem),
        ]
        zero = jnp.zeros((L,), jnp.float32)

        @plsc.parallel_loop(0, NPK // L, unroll=8)
        def _(i):
            acc[pl.ds(i * L, L)] = zero

        for cp in cps:
            cp.wait()

        @plsc.parallel_loop(0, steps, unroll=4)
        def _(i):
            sl = pl.ds(i * L, L)
            sd16 = sdv[sl]
            s16 = lax.shift_right_logical(sd16, 14)
            d16 = sd16 & 16383
            n16 = nrmv[sl]
            for k in range(K):
                g = plsc.load_gather(hv, [s16 + (k * NP)])
                plsc.addupdate_scatter(acc, [d16 + (k * NP)], g * n16)

        pltpu.async_copy(acc, out_hbm.at[w], sem).wait()

    return agg_kernel


def _tc_prep(xT, W1T, degp, NP):
    def body(x_ref, w_ref, degp_ref, hp_ref, dinv_ref):
        deg = lax.max(jnp.sum(degp_ref[...], axis=0), 1.0)
        dinv_ref[...] = lax.rsqrt(deg)
        hp_ref[...] = jnp.dot(w_ref[...], x_ref[...],
                              preferred_element_type=jnp.float32)

    return pl.pallas_call(
        body,
        out_shape=(
            jax.ShapeDtypeStruct((W1T.shape[0], NP), jnp.float32),
            jax.ShapeDtypeStruct((NP,), jnp.float32),
        ),
    )(xT, W1T, degp)


def _tc_layer(p, b, WnT, NP, K, last=False, bn=None):
    if last:
        def body(p_ref, b_ref, w_ref, bn_ref, o_ref, h_ref):
            agg = jnp.sum(p_ref[...], axis=0) + b_ref[...]
            h = jnp.tanh(agg)
            h_ref[...] = h
            o_ref[...] = jnp.dot(w_ref[...], h,
                                 preferred_element_type=jnp.float32) + bn_ref[...]

        return pl.pallas_call(
            body,
            out_shape=(
                jax.ShapeDtypeStruct((WnT.shape[0], NP), jnp.float32),
                jax.ShapeDtypeStruct((K, NP), jnp.float32),
            ),
        )(p, b, WnT, bn)

    def body(p_ref, b_ref, w_ref, o_ref):
        agg = jnp.sum(p_ref[...], axis=0) + b_ref[...]
        h = jnp.tanh(agg)
        o_ref[...] = jnp.dot(w_ref[...], h,
                             preferred_element_type=jnp.float32)

    return pl.pallas_call(
        body,
        out_shape=jax.ShapeDtypeStruct((WnT.shape[0], NP), jnp.float32),
    )(p, b, WnT)


def kernel(x, edge_index, W1, b1, W2, b2, W3, b3, Wc, bc):
    N, D = x.shape
    E = edge_index.shape[1]

    NP = ((N + 511) // 512) * 512
    ET = (((E + N) + 511) // 512) * 512

    xT = jnp.zeros((D, NP), x.dtype).at[:, :N].set(x.T)
    loop = jnp.arange(N, dtype=jnp.int32)
    pad = jnp.full((ET - E - N,), N, jnp.int32)
    srcf = jnp.concatenate([edge_index[0], loop, pad])
    dstf = jnp.concatenate([edge_index[1], loop, pad])
    sdf = srcf * 16384 + dstf

    degp = _make_deg(ET, NP)(sdf)
    hp1, dinv = _tc_prep(xT, W1.T, degp, NP)

    K1 = W1.shape[1]
    K2 = W2.shape[1]
    K3 = W3.shape[1]

    p1, nrm = _make_agg_norm(ET, NP, K1)(hp1.reshape(-1), sdf, dinv)
    p1 = p1.reshape(NW, K1, NP)
    hp2 = _tc_layer(p1, b1[:, None], W2.T, NP, K1)

    p2 = _make_agg(ET, NP, K2)(hp2.reshape(-1), sdf,
                               nrm).reshape(NW, K2, NP)
    hp3 = _tc_layer(p2, b2[:, None], W3.T, NP, K2)

    p3 = _make_agg(ET, NP, K3)(hp3.reshape(-1), sdf,
                               nrm).reshape(NW, K3, NP)
    outT, hT = _tc_layer(p3, b3[:, None], Wc.T, NP, K3, last=True,
                         bn=bc[:, None])

    return (outT.T[:N], hT.T[:N])
